# baseline (speedup 1.0000x reference)
import functools

import jax
import jax.numpy as jnp
from jax import lax
from jax.experimental import pallas as pl
from jax.experimental.pallas import tpu as pltpu
from jax.experimental.pallas import tpu_sc as plsc


def _make_gather(N, R, D, F, V):
    info = plsc.get_sparse_core_info()
    NC, NS, L = info.num_cores, info.num_subcores, info.num_lanes
    NW = NC * NS
    n_per_w = N // NW
    assert N % NW == 0 and D == L
    CHUNK = 3328
    NCHUNK = n_per_w // CHUNK
    assert n_per_w % CHUNK == 0
    PERIOD = 208
    assert n_per_w % PERIOD == 0 and PERIOD % F == 0 and PERIOD % L == 0

    mesh = plsc.VectorSubcoreMesh(core_axis_name="c", subcore_axis_name="s")

    @functools.partial(
        pl.kernel,
        mesh=mesh,
        compiler_params=pltpu.CompilerParams(use_tc_tiling_on_sc=False),
        out_type=jax.ShapeDtypeStruct((N, D), jnp.float32),
        scratch_types=[
            pltpu.VMEM((n_per_w,), jnp.int32),
            pltpu.VMEM((2, CHUNK, D), jnp.float32),
            pltpu.SemaphoreType.DMA,
            pltpu.SemaphoreType.DMA,
        ],
    )
    def gather_kernel(idx_hbm, tab_hbm, out_hbm, idx_v, rows_v, gsem, osem):
        wid = lax.axis_index("s") * NC + lax.axis_index("c")
        base = wid * n_per_w

        pltpu.sync_copy(idx_hbm.at[pl.ds(base, n_per_w)], idx_v)

        iota = lax.iota(jnp.int32, L)
        offs = [((j * L + iota) % F) * V for j in range(PERIOD // L)]

        def adjust(g, carry):
            for j in range(PERIOD // L):
                s = g * PERIOD + j * L
                idx_v[pl.ds(s, L)] = idx_v[pl.ds(s, L)] + offs[j]
            return carry

        lax.fori_loop(0, n_per_w // PERIOD, adjust, 0)

        def fire_gather(c, slot):
            return pltpu.async_copy(
                tab_hbm.at[idx_v.at[pl.ds(c * CHUNK, CHUNK)]],
                rows_v.at[slot],
                gsem,
            )

        g_cp = fire_gather(0, 0)
        o_cp = [None, None]
        for c in range(NCHUNK):
            s = c % 2
            g_cp.wait()
            if c + 1 < NCHUNK:
                s2 = (c + 1) % 2
                if o_cp[s2] is not None:
                    o_cp[s2].wait()
                g_cp = fire_gather(c + 1, s2)
            o_cp[s] = pltpu.async_copy(
                rows_v.at[s],
                out_hbm.at[pl.ds(base + c * CHUNK, CHUNK)],
                osem,
            )
        for s in range(2):
            if o_cp[s] is not None:
                o_cp[s].wait()

    return gather_kernel


def kernel(indices, tables):
    B, F = indices.shape
    _, V, D = tables.shape
    N = B * F
    idx_flat = indices.reshape(N)
    tab_flat = tables.reshape(F * V, D)
    out = _make_gather(N, F * V, D, F, V)(idx_flat, tab_flat)
    return out.reshape(B, F, D)


# speedup vs baseline: 6.7207x; 6.7207x over previous
import functools

import jax
import jax.numpy as jnp
from jax import lax
from jax.experimental import pallas as pl
from jax.experimental.pallas import tpu as pltpu
from jax.experimental.pallas import tpu_sc as plsc


def _make_pane_gather(F, D, V, B):
    info = plsc.get_sparse_core_info()
    NC, NS, L = info.num_cores, info.num_subcores, info.num_lanes
    NW = NC * NS
    P = F * D
    PW = P // NW
    assert P % NW == 0 and D == L
    HB = B // 2

    mesh = plsc.VectorSubcoreMesh(core_axis_name="c", subcore_axis_name="s")

    @functools.partial(
        pl.kernel,
        mesh=mesh,
        compiler_params=pltpu.CompilerParams(
            use_tc_tiling_on_sc=True, needs_layout_passes=False
        ),
        out_type=jax.ShapeDtypeStruct((P, B), jnp.float32),
        scratch_types=[
            pltpu.VMEM((V,), jnp.float32),
            pltpu.VMEM((HB,), jnp.int32),
            pltpu.VMEM((2, HB), jnp.float32),
            pltpu.SemaphoreType.DMA,
            pltpu.SemaphoreType.DMA,
        ],
    )
    def pane_kernel(idx_hbm, tab_hbm, out_hbm, pane_v, idx_v, out_v, psem, osem):
        wid = lax.axis_index("s") * NC + lax.axis_index("c")

        for j in range(PW):
            p = wid * PW + j
            f = p // D
            pane_cp = pltpu.async_copy(tab_hbm.at[p], pane_v, psem)
            o_cp = None
            for h in range(2):
                pltpu.sync_copy(idx_hbm.at[f, pl.ds(h * HB, HB)], idx_v)
                if h == 0:
                    pane_cp.wait()

                def gather_block(i, carry):
                    iv = idx_v[pl.ds(i * L, L)]
                    out_v[h, pl.ds(i * L, L)] = plsc.load_gather(pane_v, [iv])
                    return carry

                lax.fori_loop(0, HB // L, gather_block, 0)
                if o_cp is not None:
                    o_cp.wait()
                o_cp = pltpu.async_copy(
                    out_v.at[h], out_hbm.at[p, pl.ds(h * HB, HB)], osem
                )
            o_cp.wait()

    return pane_kernel


def kernel(indices, tables):
    B, F = indices.shape
    _, V, D = tables.shape
    idx_t = indices.T
    tab_panes = tables.transpose(0, 2, 1).reshape(F * D, V)
    out_t = _make_pane_gather(F, D, V, B)(idx_t, tab_panes)
    return out_t.reshape(F, D, B).transpose(2, 0, 1)


# speedup vs baseline: 12.1439x; 1.8070x over previous
import functools

import jax
import jax.numpy as jnp
from jax import lax
from jax.experimental import pallas as pl
from jax.experimental.pallas import tpu as pltpu
from jax.experimental.pallas import tpu_sc as plsc


def _make_pane_gather(F, D, V, B):
    info = plsc.get_sparse_core_info()
    NC, NS, L = info.num_cores, info.num_subcores, info.num_lanes
    NW = NC * NS
    P = F * D
    PW = P // NW
    assert P % NW == 0 and D == L
    QB = B // 4
    U = 8
    VH = (V // 2) // 128 * 128

    mesh = plsc.VectorSubcoreMesh(core_axis_name="c", subcore_axis_name="s")

    @functools.partial(
        pl.kernel,
        mesh=mesh,
        compiler_params=pltpu.CompilerParams(
            use_tc_tiling_on_sc=True, needs_layout_passes=False
        ),
        out_type=jax.ShapeDtypeStruct((P, B), jnp.float32),
        scratch_types=[
            pltpu.VMEM((V,), jnp.float32),
            pltpu.VMEM((B,), jnp.int32),
            pltpu.VMEM((2, QB), jnp.float32),
            pltpu.SemaphoreType.DMA,
            pltpu.SemaphoreType.DMA,
        ],
    )
    def pane_kernel(idx_hbm, tab_hbm, out_hbm, pane_v, idx_v, out_v, psem, osem):
        wid = lax.axis_index("s") * NC + lax.axis_index("c")

        o_cp = [None, None]
        for j in range(PW):
            p = wid * PW + j
            f = p // D
            pane_cps = [pltpu.async_copy(tab_hbm.at[p], pane_v, psem)]
            @pl.when(jnp.logical_or(p % D == 0, j == 0))
            def _load_idx():
                pltpu.sync_copy(idx_hbm.at[f, pl.ds(0, B)], idx_v)
            for cp in pane_cps:
                cp.wait()

            for q in range(4):
                s = q % 2

                def gather_block(i, carry):
                    b0 = i * L * U
                    ivs = [
                        idx_v[pl.ds(q * QB + b0 + k * L, L)] for k in range(U)
                    ]
                    res = [plsc.load_gather(pane_v, [iv]) for iv in ivs]
                    for k in range(U):
                        out_v[s, pl.ds(b0 + k * L, L)] = res[k]
                    return carry

                if o_cp[s] is not None:
                    o_cp[s].wait()
                lax.fori_loop(0, QB // (L * U), gather_block, 0)
                o_cp[s] = pltpu.async_copy(
                    out_v.at[s], out_hbm.at[p, pl.ds(q * QB, QB)], osem
                )
        o_cp[0].wait()
        o_cp[1].wait()

    return pane_kernel


def kernel(indices, tables):
    B, F = indices.shape
    _, V, D = tables.shape
    idx_t = indices.T
    tab_panes = tables.transpose(0, 2, 1).reshape(F * D, V)
    out_t = _make_pane_gather(F, D, V, B)(idx_t, tab_panes)
    return out_t.reshape(F, D, B).transpose(2, 0, 1)
